# Initial kernel scaffold; baseline (speedup 1.0000x reference)
import jax
import jax.numpy as jnp
from jax.experimental import pallas as pl


def kernel(sh_vectors, cutoffs, receivers, inv_avg_num_neighbors):
    raise NotImplementedError("write your pallas kernel here")


# speedup vs baseline: 5.3688x; 5.3688x over previous
import functools

import jax
import jax.numpy as jnp
from jax import lax
from jax.experimental import pallas as pl
from jax.experimental.pallas import tpu as pltpu
from jax.experimental.pallas import tpu_sc as plsc

_N_NODES = 100000
_N_PAD = 102400
_SH = 16
_E = 3200000
_NC = 2
_NS = 16
_NW = _NC * _NS
_E_PER_W = _E // _NW
_CHUNK = 800
_NCHUNK = _E_PER_W // _CHUNK
_ROWS_PER_TILE = _N_PAD // _NS
_ZROWS = 800
_SBATCH = 100


def _sc_body(sh_hbm, cut_hbm, recv_hbm, out_hbm, sh_v, cut_v, idx_v, acc):
    cid = lax.axis_index("c")
    sid = lax.axis_index("s")
    wid = sid * _NC + cid

    def _zrow(i, carry):
        sh_v[i, :] = jnp.zeros((_SH,), jnp.float32)
        return carry

    lax.fori_loop(0, _ZROWS, _zrow, None)
    for j in range(_ROWS_PER_TILE // _ZROWS):
        r0 = sid * _ROWS_PER_TILE + j * _ZROWS
        pltpu.sync_copy(sh_v.at[pl.ds(0, _ZROWS)], acc.at[pl.ds(r0, _ZROWS)])
    plsc.subcore_barrier()

    def _chunk(i, carry):
        base = wid * _E_PER_W + i * _CHUNK
        pltpu.sync_copy(sh_hbm.at[pl.ds(base, _CHUNK)], sh_v)
        pltpu.sync_copy(cut_hbm.at[pl.ds(base, _CHUNK)], cut_v)
        pltpu.sync_copy(
            recv_hbm.at[pl.ds(base // _SBATCH, _CHUNK // _SBATCH)], idx_v
        )

        def _mul16(g, c2):
            cvec = cut_v[pl.ds(g * _SH, _SH)]
            for j in range(_SH):
                e = g * _SH + j
                sh_v[e, :] = sh_v[e, :] * cvec[j]
            return c2

        lax.fori_loop(0, _CHUNK // _SH, _mul16, None)
        for j in range(_CHUNK // _SBATCH):
            pltpu.sync_copy(
                sh_v.at[pl.ds(j * _SBATCH, _SBATCH)],
                acc.at[idx_v.at[j]],
                add=True,
            )
        return carry

    lax.fori_loop(0, _NCHUNK, _chunk, None)
    plsc.subcore_barrier()

    for j in range(_ROWS_PER_TILE // _ZROWS):
        r0 = sid * _ROWS_PER_TILE + j * _ZROWS
        pltpu.sync_copy(acc.at[pl.ds(r0, _ZROWS)], sh_v.at[pl.ds(0, _ZROWS)])
        pltpu.sync_copy(
            sh_v.at[pl.ds(0, _ZROWS)],
            out_hbm.at[pl.ds(cid * _N_PAD + r0, _ZROWS)],
        )


_sc_scatter = functools.partial(
    pl.kernel,
    mesh=plsc.VectorSubcoreMesh(core_axis_name="c", subcore_axis_name="s"),
    out_type=jax.ShapeDtypeStruct((_NC * _N_PAD, _SH), jnp.float32),
    compiler_params=pltpu.CompilerParams(use_tc_tiling_on_sc=False),
    scratch_types=[
        pltpu.VMEM((_CHUNK, _SH), jnp.float32),
        pltpu.VMEM((_CHUNK,), jnp.float32),
        pltpu.VMEM((_CHUNK // _SBATCH, _SBATCH), jnp.int32),
        pltpu.VMEM_SHARED((_N_PAD, _SH), jnp.float32),
    ],
)(_sc_body)

_RB = 512
_CB = (_N_PAD * _SH) // _RB
_GB = 64


def _combine_body(inv_ref, p_ref, o_ref):
    o_ref[...] = (p_ref[0] + p_ref[1]) * inv_ref[0]


def kernel(sh_vectors, cutoffs, receivers, inv_avg_num_neighbors):
    cut = cutoffs.reshape(_E)
    recv = receivers.astype(jnp.int32).reshape(_E // _SBATCH, _SBATCH)
    part = _sc_scatter(sh_vectors, cut, recv)
    inv_arr = jnp.asarray(inv_avg_num_neighbors, jnp.float32).reshape(1)
    out = pl.pallas_call(
        _combine_body,
        grid=(_RB // _GB,),
        in_specs=[
            pl.BlockSpec(memory_space=pltpu.SMEM),
            pl.BlockSpec((_NC, _GB, _CB), lambda i: (0, i, 0)),
        ],
        out_specs=pl.BlockSpec((_GB, _CB), lambda i: (i, 0)),
        out_shape=jax.ShapeDtypeStruct((_RB, _CB), jnp.float32),
    )(inv_arr, part.reshape(_NC, _RB, _CB))
    return out.reshape(_N_PAD, _SH)[:_N_NODES]


# speedup vs baseline: 6.5496x; 1.2200x over previous
import functools

import jax
import jax.numpy as jnp
from jax import lax
from jax.experimental import pallas as pl
from jax.experimental.pallas import tpu as pltpu
from jax.experimental.pallas import tpu_sc as plsc

_N_NODES = 100000
_N_PAD = 102144
_SH = 16
_E = 3200000
_NC = 2
_NS = 16
_NW = _NC * _NS
_E_PER_W = _E // _NW
_CHUNK = 800
_NCHUNK = _E_PER_W // _CHUNK
_NBUF = 2
_ROWS_PER_TILE = _N_PAD // _NS
_ZROWS = 456


def _start_in(b, i, wid, sh_hbm_r, cut_hbm, recv_hbm, sh_v, cut_v, idx_v, sems):
    base = wid * _E_PER_W + i * _CHUNK
    pltpu.async_copy(sh_hbm_r.at[pl.ds(base, _CHUNK)], sh_v.at[b], sems.at[b])
    pltpu.async_copy(cut_hbm.at[pl.ds(base, _CHUNK)], cut_v.at[b], sems.at[b])
    pltpu.async_copy(recv_hbm.at[pl.ds(base, _CHUNK)], idx_v.at[b], sems.at[b])


def _wait_in(b, sh_hbm_r, cut_hbm, recv_hbm, sh_v, cut_v, idx_v, sems):
    pltpu.make_async_copy(sh_hbm_r.at[pl.ds(0, _CHUNK)], sh_v.at[b], sems.at[b]).wait()
    pltpu.make_async_copy(cut_hbm.at[pl.ds(0, _CHUNK)], cut_v.at[b], sems.at[b]).wait()
    pltpu.make_async_copy(recv_hbm.at[pl.ds(0, _CHUNK)], idx_v.at[b], sems.at[b]).wait()


def _sc_body(sh_hbm, cut_hbm, recv_hbm, out_hbm, sh_v, cut_v, idx_v, acc, sems):
    cid = lax.axis_index("c")
    sid = lax.axis_index("s")
    wid = sid * _NC + cid
    sh_hbm_r = sh_hbm
    out_hbm_r = out_hbm

    def _zrow(i, carry):
        sh_v[0, i, :] = jnp.zeros((_SH,), jnp.float32)
        return carry

    lax.fori_loop(0, _ZROWS, _zrow, None)
    for j in range(_ROWS_PER_TILE // _ZROWS):
        r0 = sid * _ROWS_PER_TILE + j * _ZROWS
        pltpu.sync_copy(sh_v.at[0, pl.ds(0, _ZROWS)], acc.at[pl.ds(r0, _ZROWS)])
    plsc.subcore_barrier()

    for b in range(_NBUF):
        _start_in(b, b, wid, sh_hbm_r, cut_hbm, recv_hbm, sh_v, cut_v, idx_v, sems)

    def _process(i, b):
        _wait_in(b, sh_hbm_r, cut_hbm, recv_hbm, sh_v, cut_v, idx_v, sems)

        def _mul16(g, c2):
            cvec = cut_v[b, pl.ds(g * _SH, _SH)]
            for j in range(_SH):
                e = g * _SH + j
                sh_v[b, e, :] = sh_v[b, e, :] * cvec[j]
            return c2

        lax.fori_loop(0, _CHUNK // _SH, _mul16, None)
        pltpu.sync_copy(sh_v.at[b], acc.at[idx_v.at[b]], add=True)

        @pl.when(i + _NBUF < _NCHUNK)
        def _refill():
            _start_in(b, i + _NBUF, wid, sh_hbm_r, cut_hbm, recv_hbm,
                      sh_v, cut_v, idx_v, sems)

    def _pair(k, carry):
        for b in range(_NBUF):
            _process(k * _NBUF + b, b)
        return carry

    lax.fori_loop(0, _NCHUNK // _NBUF, _pair, None)
    for r in range(_NCHUNK - (_NCHUNK // _NBUF) * _NBUF):
        _process(_NCHUNK - 1 + r, (_NCHUNK - 1 + r) % _NBUF)
    plsc.subcore_barrier()

    for j in range(_ROWS_PER_TILE // _ZROWS):
        r0 = sid * _ROWS_PER_TILE + j * _ZROWS
        pltpu.sync_copy(acc.at[pl.ds(r0, _ZROWS)], sh_v.at[0, pl.ds(0, _ZROWS)])
        pltpu.sync_copy(
            sh_v.at[0, pl.ds(0, _ZROWS)],
            out_hbm_r.at[pl.ds(cid * _N_PAD + r0, _ZROWS)],
        )


_sc_scatter = functools.partial(
    pl.kernel,
    mesh=plsc.VectorSubcoreMesh(core_axis_name="c", subcore_axis_name="s"),
    out_type=jax.ShapeDtypeStruct((_NC * _N_PAD, _SH), jnp.float32),
    compiler_params=pltpu.CompilerParams(use_tc_tiling_on_sc=False),
    scratch_types=[
        pltpu.VMEM((_NBUF, _CHUNK, _SH), jnp.float32),
        pltpu.VMEM((_NBUF, _CHUNK), jnp.float32),
        pltpu.VMEM((_NBUF, _CHUNK), jnp.int32),
        pltpu.VMEM_SHARED((_N_PAD, _SH), jnp.float32),
        pltpu.SemaphoreType.DMA((_NBUF,)),
    ],
)(_sc_body)

_RB = 512
_CB = _N_PAD * _SH // _RB
_GB = 64


def _combine_body(inv_ref, p_ref, o_ref):
    o_ref[...] = (p_ref[0] + p_ref[1]) * inv_ref[0]


def kernel(sh_vectors, cutoffs, receivers, inv_avg_num_neighbors):
    cut = cutoffs.reshape(_E)
    recv = receivers.astype(jnp.int32)
    part = _sc_scatter(sh_vectors, cut, recv)
    inv_arr = jnp.asarray(inv_avg_num_neighbors, jnp.float32).reshape(1)
    out = pl.pallas_call(
        _combine_body,
        grid=(_RB // _GB,),
        in_specs=[
            pl.BlockSpec(memory_space=pltpu.SMEM),
            pl.BlockSpec((_NC, _GB, _CB), lambda i: (0, i, 0)),
        ],
        out_specs=pl.BlockSpec((_GB, _CB), lambda i: (i, 0)),
        out_shape=jax.ShapeDtypeStruct((_RB, _CB), jnp.float32),
    )(inv_arr, part.reshape(_NC, _RB, _CB))
    return out.reshape(_N_PAD, _SH)[:_N_NODES]
